# Initial kernel scaffold; baseline (speedup 1.0000x reference)
import jax
import jax.numpy as jnp
from jax.experimental import pallas as pl


def kernel(x, edge_index, W_gat, att_src, att_dst, b_gat, W_comp, b_comp, W_ih, W_hh, b_ih, b_hh, W_opt, b_opt):
    raise NotImplementedError("write your pallas kernel here")


# speedup vs baseline: 12.5483x; 12.5483x over previous
import functools

import jax
import jax.numpy as jnp
from jax import lax
from jax.experimental import pallas as pl
from jax.experimental.pallas import tpu as pltpu
from jax.experimental.pallas import tpu_sc as plsc

N = 10000
E = 320000
D = 128
H = 8

NPAD = 10240
NB = NPAD // 256
ETOT = E + N
C = 256
NS = 16
NCORE = 2
CHUNKS = -(-ETOT // (NS * C))
EPAD = NS * C * CHUNKS
ER = EPAD // 128
TA = EPAD // NS
NROW_T = NPAD // NS


def _head_body(x_ref, wg_ref, vs_ref, vd_ref, xp_ref, us_ref, ud_ref):
    xb = x_ref[...]
    wg = wg_ref[...]
    for h in range(H):
        wh = wg[h * D:(h + 1) * D, :]
        xp_ref[h] = lax.dot_general(
            xb, wh, (((1,), (1,)), ((), ())),
            preferred_element_type=jnp.float32)
    us_ref[...] = jnp.dot(xb, vs_ref[...], preferred_element_type=jnp.float32)
    ud_ref[...] = jnp.dot(xb, vd_ref[...], preferred_element_type=jnp.float32)


def _tail_body(og_ref, den_ref, x_ref, wc_ref, bg_ref, bc_ref, wih_ref,
               whh_ref, bih_ref, bhh_ref, wopt_ref, bopt_ref, o_ref):
    xb = x_ref[...]
    den = den_ref[...]
    wc = wc_ref[...]
    bg = bg_ref[...]
    acc = jnp.zeros((256, D), jnp.float32)
    for h in range(H):
        g = og_ref[h] / (den[:, h:h + 1] + 1e-16)
        g = jnp.maximum(g + bg[:, h * D:(h + 1) * D], 0.0)
        acc = acc + lax.dot_general(
            g, wc[:, h * D:(h + 1) * D], (((1,), (1,)), ((), ())),
            preferred_element_type=jnp.float32)
    m = jnp.maximum(acc + bc_ref[...], 0.0)
    gi = lax.dot_general(m, wih_ref[...], (((1,), (1,)), ((), ())),
                         preferred_element_type=jnp.float32) + bih_ref[...]
    gh = lax.dot_general(xb, whh_ref[...], (((1,), (1,)), ((), ())),
                         preferred_element_type=jnp.float32) + bhh_ref[...]
    r = jax.nn.sigmoid(gi[:, 0:D] + gh[:, 0:D])
    z = jax.nn.sigmoid(gi[:, D:2 * D] + gh[:, D:2 * D])
    n = jnp.tanh(gi[:, 2 * D:] + r * gh[:, 2 * D:])
    hv = jnp.tanh((1.0 - z) * n + z * xb)
    o_ref[...] = lax.dot_general(
        hv, wopt_ref[...], (((1,), (1,)), ((), ())),
        preferred_element_type=jnp.float32) + bopt_ref[...]


def _sc_body(xp_hbm, us_hbm, ud_hbm, src_hbm, dst_hbm,
             outg_hbm, den_hbm, s_hbm,
             rows_vm, s_vm, us_vm, ud_vm, idx2, dst2,
             out_sp, sem_a, sem_b):
    c = lax.axis_index("c")
    sid = lax.axis_index("s")
    tile_e0 = sid * TA
    tile_r0 = sid * (TA // 128)
    zrow = sid * NROW_T

    def _zero_rows(_i, _):
        for j in range(8):
            rows_vm[_i, pl.ds(j * 16, 16)] = jnp.zeros((16,), jnp.float32)
        return _

    def _zero_out_sp():
        pltpu.sync_copy(rows_vm, out_sp.at[pl.ds(zrow, 256)])
        pltpu.sync_copy(rows_vm, out_sp.at[pl.ds(zrow + 256, 256)])
        pltpu.sync_copy(rows_vm.at[pl.ds(0, 128)],
                        out_sp.at[pl.ds(zrow + 512, 128)])

    lax.fori_loop(0, C, _zero_rows, 0)
    _zero_out_sp()
    plsc.subcore_barrier()

    def _phase_a(g, _):
        e0 = tile_e0 + g * C
        r0 = tile_r0 + g * 2
        pltpu.sync_copy(src_hbm.at[pl.ds(r0, 2)], idx2)
        pltpu.sync_copy(dst_hbm.at[pl.ds(r0, 2)], dst2)
        d1 = pltpu.async_copy(us_hbm.at[idx2.at[0]], us_vm.at[pl.ds(0, 128)], sem_a)
        d2 = pltpu.async_copy(us_hbm.at[idx2.at[1]], us_vm.at[pl.ds(128, 128)], sem_a)
        d3 = pltpu.async_copy(ud_hbm.at[dst2.at[0]], ud_vm.at[pl.ds(0, 128)], sem_b)
        d4 = pltpu.async_copy(ud_hbm.at[dst2.at[1]], ud_vm.at[pl.ds(128, 128)], sem_b)
        d1.wait()
        d2.wait()
        d3.wait()
        d4.wait()

        def _edge(e, _):
            v = us_vm[e] + ud_vm[e]
            v = jnp.where(v > 0.0, v, 0.2 * v)
            s16 = jnp.exp(v)
            s_vm[e] = s16
            rows_vm[e, pl.ds(0, 16)] = s16
            return _

        lax.fori_loop(0, C, _edge, 0)
        pltpu.sync_copy(rows_vm.at[pl.ds(0, 128)], out_sp.at[dst2.at[0]], add=True)
        pltpu.sync_copy(rows_vm.at[pl.ds(128, 128)], out_sp.at[dst2.at[1]], add=True)
        pltpu.sync_copy(s_vm, s_hbm.at[c, pl.ds(e0, C)])
        return _

    lax.fori_loop(0, CHUNKS, _phase_a, 0)
    plsc.subcore_barrier()

    @pl.when(c == 0)
    def _():
        pltpu.sync_copy(out_sp.at[pl.ds(zrow, 256)], den_hbm.at[pl.ds(zrow, 256)])
        pltpu.sync_copy(out_sp.at[pl.ds(zrow + 256, 256)],
                        den_hbm.at[pl.ds(zrow + 256, 256)])
        pltpu.sync_copy(out_sp.at[pl.ds(zrow + 512, 128)],
                        den_hbm.at[pl.ds(zrow + 512, 128)])

    for hh in range(H // NCORE):
        h = c * (H // NCORE) + hh
        lax.fori_loop(0, C, _zero_rows, 0)
        _zero_out_sp()
        plsc.subcore_barrier()

        hoff = h * NPAD

        def _phase_b(g, _):
            e0 = tile_e0 + g * C
            r0 = tile_r0 + g * 2
            pltpu.sync_copy(src_hbm.at[pl.ds(r0, 2)], idx2)
            pltpu.sync_copy(dst_hbm.at[pl.ds(r0, 2)], dst2)
            for j in range(2):
                for k in range(8):
                    idx2[j, pl.ds(k * 16, 16)] = (
                        idx2[j, pl.ds(k * 16, 16)] + hoff)
            d1 = pltpu.async_copy(xp_hbm.at[idx2.at[0]],
                                  rows_vm.at[pl.ds(0, 128)], sem_a)
            d2 = pltpu.async_copy(xp_hbm.at[idx2.at[1]],
                                  rows_vm.at[pl.ds(128, 128)], sem_a)
            pltpu.sync_copy(s_hbm.at[c, pl.ds(e0, C)], s_vm)
            d1.wait()
            d2.wait()

            def _edge(e, _):
                sv = s_vm[e]
                bc = lax.gather(
                    sv, jnp.full((16, 1), h, jnp.int32),
                    lax.GatherDimensionNumbers(
                        offset_dims=(), collapsed_slice_dims=(0,),
                        start_index_map=(0,)),
                    slice_sizes=(1,),
                    mode=lax.GatherScatterMode.PROMISE_IN_BOUNDS)
                for j in range(8):
                    rows_vm[e, pl.ds(j * 16, 16)] = (
                        rows_vm[e, pl.ds(j * 16, 16)] * bc)
                return _

            lax.fori_loop(0, C, _edge, 0)
            pltpu.sync_copy(rows_vm.at[pl.ds(0, 128)],
                            out_sp.at[dst2.at[0]], add=True)
            pltpu.sync_copy(rows_vm.at[pl.ds(128, 128)],
                            out_sp.at[dst2.at[1]], add=True)
            return _

        lax.fori_loop(0, CHUNKS, _phase_b, 0)
        plsc.subcore_barrier()
        pltpu.sync_copy(out_sp.at[pl.ds(zrow, 256)],
                        outg_hbm.at[h, pl.ds(zrow, 256)])
        pltpu.sync_copy(out_sp.at[pl.ds(zrow + 256, 256)],
                        outg_hbm.at[h, pl.ds(zrow + 256, 256)])
        pltpu.sync_copy(out_sp.at[pl.ds(zrow + 512, 128)],
                        outg_hbm.at[h, pl.ds(zrow + 512, 128)])
        plsc.subcore_barrier()


@functools.partial(
    pl.kernel,
    out_type=(
        jax.ShapeDtypeStruct((H, NPAD, D), jnp.float32),
        jax.ShapeDtypeStruct((NPAD, D), jnp.float32),
        jax.ShapeDtypeStruct((NCORE, EPAD, 16), jnp.float32),
    ),
    mesh=plsc.VectorSubcoreMesh(
        core_axis_name="c", subcore_axis_name="s",
        num_cores=NCORE, num_subcores=NS),
    compiler_params=pltpu.CompilerParams(use_tc_tiling_on_sc=False),
    scratch_types=[
        pltpu.VMEM((C, D), jnp.float32),
        pltpu.VMEM((C, 16), jnp.float32),
        pltpu.VMEM((C, 16), jnp.float32),
        pltpu.VMEM((C, 16), jnp.float32),
        pltpu.VMEM((2, 128), jnp.int32),
        pltpu.VMEM((2, 128), jnp.int32),
        pltpu.VMEM_SHARED((NPAD, D), jnp.float32),
        pltpu.SemaphoreType.DMA,
        pltpu.SemaphoreType.DMA,
    ],
)
def _sc_kernel(xp_hbm, us_hbm, ud_hbm, src_hbm, dst_hbm,
               outg_hbm, den_hbm, s_hbm,
               rows_vm, s_vm, us_vm, ud_vm, idx2, dst2,
               out_sp, sem_a, sem_b):
    _sc_body(xp_hbm, us_hbm, ud_hbm, src_hbm, dst_hbm,
             outg_hbm, den_hbm, s_hbm,
             rows_vm, s_vm, us_vm, ud_vm, idx2, dst2,
             out_sp, sem_a, sem_b)


def kernel(x, edge_index, W_gat, att_src, att_dst, b_gat, W_comp, b_comp,
           W_ih, W_hh, b_ih, b_hh, W_opt, b_opt):
    f32 = jnp.float32
    x_pad = jnp.zeros((NPAD, D), f32).at[:N].set(x)

    wg3 = W_gat.reshape(H, D, D)
    vsrc = jnp.einsum("hdk,hd->kh", wg3, att_src)
    vdst = jnp.einsum("hdk,hd->kh", wg3, att_dst)
    vsrc2 = jnp.concatenate([vsrc, vsrc], axis=1)
    vdst2 = jnp.concatenate([vdst, vdst], axis=1)

    loop = jnp.arange(N, dtype=jnp.int32)
    src = jnp.concatenate([edge_index[0].astype(jnp.int32), loop])
    dst = jnp.concatenate([edge_index[1].astype(jnp.int32), loop])
    pad = jnp.full((EPAD - ETOT,), N, jnp.int32)
    src2d = jnp.concatenate([src, pad]).reshape(ER, 128)
    dst2d = jnp.concatenate([dst, pad]).reshape(ER, 128)

    xp_t, us, ud = pl.pallas_call(
        _head_body,
        grid=(NB,),
        in_specs=[
            pl.BlockSpec((256, D), lambda i: (i, 0)),
            pl.BlockSpec((H * D, D), lambda i: (0, 0)),
            pl.BlockSpec((D, 16), lambda i: (0, 0)),
            pl.BlockSpec((D, 16), lambda i: (0, 0)),
        ],
        out_specs=[
            pl.BlockSpec((H, 256, D), lambda i: (0, i, 0)),
            pl.BlockSpec((256, 16), lambda i: (i, 0)),
            pl.BlockSpec((256, 16), lambda i: (i, 0)),
        ],
        out_shape=[
            jax.ShapeDtypeStruct((H, NPAD, D), f32),
            jax.ShapeDtypeStruct((NPAD, 16), f32),
            jax.ShapeDtypeStruct((NPAD, 16), f32),
        ],
    )(x_pad, W_gat, vsrc2, vdst2)

    xp_flat = xp_t.reshape(H * NPAD, D)
    out_gat, denom, _ = _sc_kernel(xp_flat, us, ud, src2d, dst2d)

    b2 = lambda b: b.reshape(1, -1)
    y = pl.pallas_call(
        _tail_body,
        grid=(NB,),
        in_specs=[
            pl.BlockSpec((H, 256, D), lambda i: (0, i, 0)),
            pl.BlockSpec((256, D), lambda i: (i, 0)),
            pl.BlockSpec((256, D), lambda i: (i, 0)),
            pl.BlockSpec((D, H * D), lambda i: (0, 0)),
            pl.BlockSpec((1, H * D), lambda i: (0, 0)),
            pl.BlockSpec((1, D), lambda i: (0, 0)),
            pl.BlockSpec((3 * D, D), lambda i: (0, 0)),
            pl.BlockSpec((3 * D, D), lambda i: (0, 0)),
            pl.BlockSpec((1, 3 * D), lambda i: (0, 0)),
            pl.BlockSpec((1, 3 * D), lambda i: (0, 0)),
            pl.BlockSpec((D, D), lambda i: (0, 0)),
            pl.BlockSpec((1, D), lambda i: (0, 0)),
        ],
        out_specs=pl.BlockSpec((256, D), lambda i: (i, 0)),
        out_shape=jax.ShapeDtypeStruct((NPAD, D), f32),
    )(out_gat, denom, x_pad, W_comp, b2(b_gat), b2(b_comp), W_ih, W_hh,
      b2(b_ih), b2(b_hh), W_opt, b2(b_opt))

    return y[:N]


# speedup vs baseline: 25.1017x; 2.0004x over previous
import functools

import jax
import jax.numpy as jnp
from jax import lax
from jax.experimental import pallas as pl
from jax.experimental.pallas import tpu as pltpu
from jax.experimental.pallas import tpu_sc as plsc

N = 10000
E = 320000
D = 128
H = 8

NPAD = 10240
NB = NPAD // 256
ETOT = E + N
C = 128
NS = 16
NCORE = 2
GR = 9
EPAD = 331776
ER = EPAD // 128
TA = EPAD // NS
TR = TA // 128
NGRP = TR // GR
CH_A = EPAD // (NCORE * NS * C)
NROW_T = NPAD // NS


def _head_body(x_ref, wg_ref, vs_ref, vd_ref, xp_ref, us_ref, ud_ref):
    xb = x_ref[...]
    wg = wg_ref[...]
    for h in range(H):
        wh = wg[h * D:(h + 1) * D, :]
        xp_ref[h] = lax.dot_general(
            xb, wh, (((1,), (1,)), ((), ())),
            preferred_element_type=jnp.float32)
    us_ref[...] = jnp.dot(xb, vs_ref[...], preferred_element_type=jnp.float32)
    ud_ref[...] = jnp.dot(xb, vd_ref[...], preferred_element_type=jnp.float32)


def _tail_body(og_ref, den_ref, x_ref, wc_ref, bg_ref, bc_ref, wih_ref,
               whh_ref, bih_ref, bhh_ref, wopt_ref, bopt_ref, o_ref):
    xb = x_ref[...]
    den = den_ref[0] + den_ref[1]
    wc = wc_ref[...]
    bg = bg_ref[...]
    acc = jnp.zeros((256, D), jnp.float32)
    for h in range(H):
        g = og_ref[h] / (den[:, h:h + 1] + 1e-16)
        g = jnp.maximum(g + bg[:, h * D:(h + 1) * D], 0.0)
        acc = acc + lax.dot_general(
            g, wc[:, h * D:(h + 1) * D], (((1,), (1,)), ((), ())),
            preferred_element_type=jnp.float32)
    m = jnp.maximum(acc + bc_ref[...], 0.0)
    gi = lax.dot_general(m, wih_ref[...], (((1,), (1,)), ((), ())),
                         preferred_element_type=jnp.float32) + bih_ref[...]
    gh = lax.dot_general(xb, whh_ref[...], (((1,), (1,)), ((), ())),
                         preferred_element_type=jnp.float32) + bhh_ref[...]
    r = jax.nn.sigmoid(gi[:, 0:D] + gh[:, 0:D])
    z = jax.nn.sigmoid(gi[:, D:2 * D] + gh[:, D:2 * D])
    n = jnp.tanh(gi[:, 2 * D:] + r * gh[:, 2 * D:])
    hv = jnp.tanh((1.0 - z) * n + z * xb)
    o_ref[...] = lax.dot_general(
        hv, wopt_ref[...], (((1,), (1,)), ((), ())),
        preferred_element_type=jnp.float32) + bopt_ref[...]


@functools.partial(
    pl.kernel,
    out_type=(
        jax.ShapeDtypeStruct((EPAD, 16), jnp.float32),
        jax.ShapeDtypeStruct((NCORE, NPAD, 16), jnp.float32),
    ),
    mesh=plsc.VectorSubcoreMesh(
        core_axis_name="c", subcore_axis_name="s",
        num_cores=NCORE, num_subcores=NS),
    compiler_params=pltpu.CompilerParams(use_tc_tiling_on_sc=False),
    scratch_types=[
        pltpu.VMEM((C, 16), jnp.float32),
        pltpu.VMEM((C, 16), jnp.float32),
        pltpu.VMEM((C, 16), jnp.float32),
        pltpu.VMEM((1, 128), jnp.int32),
        pltpu.VMEM((1, 128), jnp.int32),
        pltpu.VMEM_SHARED((NPAD, 16), jnp.float32),
        pltpu.SemaphoreType.DMA,
        pltpu.SemaphoreType.DMA,
    ],
)
def _sc_a(us_hbm, ud_hbm, src_hbm, dst_hbm, s_hbm, den_hbm,
          us_vm, ud_vm, s_vm, srow, drow, den_sp, sem_a, sem_b):
    c = lax.axis_index("c")
    sid = lax.axis_index("s")
    tile_r0 = c * (ER // NCORE) + sid * CH_A
    zrow = sid * NROW_T

    def _zero_svm(i, carry):
        s_vm[i, :] = jnp.zeros((16,), jnp.float32)
        return carry

    lax.fori_loop(0, C, _zero_svm, 0)
    for i in range(NROW_T // C):
        pltpu.sync_copy(s_vm, den_sp.at[pl.ds(zrow + i * C, C)])
    plsc.subcore_barrier()

    def _phase_a(g, carry):
        r0 = tile_r0 + g
        pltpu.sync_copy(src_hbm.at[pl.ds(r0, 1)], srow)
        pltpu.sync_copy(dst_hbm.at[pl.ds(r0, 1)], drow)
        d1 = pltpu.async_copy(us_hbm.at[srow.at[0]], us_vm, sem_a)
        d2 = pltpu.async_copy(ud_hbm.at[drow.at[0]], ud_vm, sem_b)
        d1.wait()
        d2.wait()

        @plsc.parallel_loop(0, C, 1, unroll=2)
        def _edge(e):
            v = us_vm[e] + ud_vm[e]
            v = jnp.where(v > 0.0, v, 0.2 * v)
            s_vm[e] = jnp.exp(v)

        pltpu.sync_copy(s_vm, den_sp.at[drow.at[0]], add=True)
        pltpu.sync_copy(s_vm, s_hbm.at[pl.ds(r0 * 128, C)])
        return carry

    lax.fori_loop(0, CH_A, _phase_a, 0)
    plsc.subcore_barrier()

    for i in range(NROW_T // C):
        pltpu.sync_copy(den_sp.at[pl.ds(zrow + i * C, C)],
                        den_hbm.at[c, pl.ds(zrow + i * C, C)])


@functools.partial(
    pl.kernel,
    out_type=jax.ShapeDtypeStruct((H, NPAD, D), jnp.float32),
    mesh=plsc.VectorSubcoreMesh(
        core_axis_name="c", subcore_axis_name="s",
        num_cores=NCORE, num_subcores=NS),
    compiler_params=pltpu.CompilerParams(use_tc_tiling_on_sc=False),
    scratch_types=[
        pltpu.VMEM((C, D), jnp.float32),
        pltpu.VMEM((C, D), jnp.float32),
        pltpu.VMEM((C, 16), jnp.float32),
        pltpu.VMEM((C, 16), jnp.float32),
        pltpu.VMEM((GR, 128), jnp.int32),
        pltpu.VMEM((GR, 128), jnp.int32),
        pltpu.VMEM((GR, 128), jnp.int32),
        pltpu.VMEM((GR, 128), jnp.int32),
        pltpu.VMEM((1, 128), jnp.int32),
        pltpu.VMEM((1, 128), jnp.int32),
        pltpu.VMEM_SHARED((NPAD, D), jnp.float32),
        pltpu.SemaphoreType.DMA,
        pltpu.SemaphoreType.DMA,
        pltpu.SemaphoreType.DMA,
        pltpu.SemaphoreType.DMA,
        pltpu.SemaphoreType.DMA,
        pltpu.SemaphoreType.DMA,
        pltpu.SemaphoreType.DMA,
    ],
)
def _sc_b(xp_hbm, src_hbm, dst_hbm, s_hbm, outg_hbm,
          rows_a, rows_b, s_a, s_b, srcg_a, srcg_b, dstg_a, dstg_b,
          idx_a, idx_b, out_sp,
          sem_x0, sem_x1, sem_s0, sem_s1, sem_w0, sem_w1, sem_g):
    c = lax.axis_index("c")
    sid = lax.axis_index("s")
    tile_r0 = sid * TR
    zrow = sid * NROW_T

    RO = (rows_a, rows_b)
    SB = (s_a, s_b)
    IX = (idx_a, idx_b)
    SX = (sem_x0, sem_x1)
    SS = (sem_s0, sem_s1)
    SW = (sem_w0, sem_w1)
    GRP = ((srcg_a, dstg_a), (srcg_b, dstg_b))

    def _zero_rows_a(i, carry):
        for j in range(8):
            rows_a[i, pl.ds(j * 16, 16)] = jnp.zeros((16,), jnp.float32)
        return carry

    def _issue_grp(g, gp):
        srcg, dstg = GRP[gp]
        r0 = tile_r0 + g * GR
        pltpu.async_copy(src_hbm.at[pl.ds(r0, GR)], srcg, sem_g)
        pltpu.async_copy(dst_hbm.at[pl.ds(r0, GR)], dstg, sem_g)

    def _wait_grp(gp):
        srcg, dstg = GRP[gp]
        pltpu.make_async_copy(src_hbm.at[pl.ds(0, GR)], srcg, sem_g).wait()
        pltpu.make_async_copy(dst_hbm.at[pl.ds(0, GR)], dstg, sem_g).wait()

    for hh in range(H // NCORE):
        h = c * (H // NCORE) + hh
        hoff = h * NPAD

        lax.fori_loop(0, C, _zero_rows_a, 0)
        for i in range(NROW_T // C):
            pltpu.sync_copy(rows_a, out_sp.at[pl.ds(zrow + i * C, C)])
        plsc.subcore_barrier()

        def _fill_idx(b, gp, k):
            srcg, _ = GRP[gp]
            for j in range(8):
                IX[b][0, pl.ds(j * 16, 16)] = (
                    srcg[k, pl.ds(j * 16, 16)] + hoff)

        def _issue_chunk(b, g, gp, k):
            _fill_idx(b, gp, k)
            pltpu.async_copy(xp_hbm.at[IX[b].at[0]], RO[b], SX[b])
            e0 = (tile_r0 + g * GR + k) * 128
            pltpu.async_copy(s_hbm.at[pl.ds(e0, C)], SB[b], SS[b])

        def _drain_scatter(b):
            _, dstg = GRP[0]
            pltpu.make_async_copy(
                RO[b], out_sp.at[dstg.at[0]], SW[b]).wait()

        def _consume(g, gp, k):
            b = (gp * GR + k) % 2
            nb = 1 - b
            if k > 0:
                _drain_scatter(nb)
            pltpu.make_async_copy(xp_hbm.at[IX[b].at[0]], RO[b], SX[b]).wait()
            pltpu.make_async_copy(s_hbm.at[pl.ds(0, C)], SB[b], SS[b]).wait()
            if k < GR - 1:
                _issue_chunk(nb, g, gp, k + 1)
            else:
                @pl.when(g < NGRP - 1)
                def _():
                    _wait_grp(1 - gp)
                    _issue_chunk(nb, g + 1, 1 - gp, 0)

            @plsc.parallel_loop(0, C, 1, unroll=2)
            def _edge(e):
                sv = SB[b][e]
                bc = lax.gather(
                    sv, jnp.full((16, 1), h, jnp.int32),
                    lax.GatherDimensionNumbers(
                        offset_dims=(), collapsed_slice_dims=(0,),
                        start_index_map=(0,)),
                    slice_sizes=(1,),
                    mode=lax.GatherScatterMode.PROMISE_IN_BOUNDS)
                for j in range(8):
                    RO[b][e, pl.ds(j * 16, 16)] = (
                        RO[b][e, pl.ds(j * 16, 16)] * bc)

            _, dstg = GRP[gp]
            sc = pltpu.make_async_copy(RO[b], out_sp.at[dstg.at[k]], SW[b])
            sc.start(add=True)
            if k == GR - 1:
                sc.wait()

                @pl.when(g < NGRP - 2)
                def _():
                    _issue_grp(g + 2, gp)

        _issue_grp(0, 0)
        _wait_grp(0)
        _issue_grp(1, 1)
        _issue_chunk(0, 0, 0, 0)

        def _group_body(g, carry):
            @pl.when((g & 1) == 0)
            def _even():
                for k in range(GR):
                    _consume(g, 0, k)

            @pl.when((g & 1) == 1)
            def _odd():
                for k in range(GR):
                    _consume(g, 1, k)

            return carry

        lax.fori_loop(0, NGRP, _group_body, 0)
        plsc.subcore_barrier()
        for i in range(NROW_T // C):
            pltpu.sync_copy(out_sp.at[pl.ds(zrow + i * C, C)],
                            outg_hbm.at[h, pl.ds(zrow + i * C, C)])
        plsc.subcore_barrier()


def kernel(x, edge_index, W_gat, att_src, att_dst, b_gat, W_comp, b_comp,
           W_ih, W_hh, b_ih, b_hh, W_opt, b_opt):
    f32 = jnp.float32
    x_pad = jnp.zeros((NPAD, D), f32).at[:N].set(x)

    wg3 = W_gat.reshape(H, D, D)
    vsrc = jnp.einsum("hdk,hd->kh", wg3, att_src)
    vdst = jnp.einsum("hdk,hd->kh", wg3, att_dst)
    vsrc2 = jnp.concatenate([vsrc, vsrc], axis=1)
    vdst2 = jnp.concatenate([vdst, vdst], axis=1)

    loop = jnp.arange(N, dtype=jnp.int32)
    src = jnp.concatenate([edge_index[0].astype(jnp.int32), loop])
    dst = jnp.concatenate([edge_index[1].astype(jnp.int32), loop])
    pad = jnp.full((EPAD - ETOT,), N, jnp.int32)
    src2d = jnp.concatenate([src, pad]).reshape(ER, 128)
    dst2d = jnp.concatenate([dst, pad]).reshape(ER, 128)

    xp_t, us, ud = pl.pallas_call(
        _head_body,
        grid=(NB,),
        in_specs=[
            pl.BlockSpec((256, D), lambda i: (i, 0)),
            pl.BlockSpec((H * D, D), lambda i: (0, 0)),
            pl.BlockSpec((D, 16), lambda i: (0, 0)),
            pl.BlockSpec((D, 16), lambda i: (0, 0)),
        ],
        out_specs=[
            pl.BlockSpec((H, 256, D), lambda i: (0, i, 0)),
            pl.BlockSpec((256, 16), lambda i: (i, 0)),
            pl.BlockSpec((256, 16), lambda i: (i, 0)),
        ],
        out_shape=[
            jax.ShapeDtypeStruct((H, NPAD, D), f32),
            jax.ShapeDtypeStruct((NPAD, 16), f32),
            jax.ShapeDtypeStruct((NPAD, 16), f32),
        ],
    )(x_pad, W_gat, vsrc2, vdst2)

    s_buf, den2 = _sc_a(us, ud, src2d, dst2d)
    xp_flat = xp_t.reshape(H * NPAD, D)
    out_gat = _sc_b(xp_flat, src2d, dst2d, s_buf)

    b2 = lambda b: b.reshape(1, -1)
    y = pl.pallas_call(
        _tail_body,
        grid=(NB,),
        in_specs=[
            pl.BlockSpec((H, 256, D), lambda i: (0, i, 0)),
            pl.BlockSpec((NCORE, 256, 16), lambda i: (0, i, 0)),
            pl.BlockSpec((256, D), lambda i: (i, 0)),
            pl.BlockSpec((D, H * D), lambda i: (0, 0)),
            pl.BlockSpec((1, H * D), lambda i: (0, 0)),
            pl.BlockSpec((1, D), lambda i: (0, 0)),
            pl.BlockSpec((3 * D, D), lambda i: (0, 0)),
            pl.BlockSpec((3 * D, D), lambda i: (0, 0)),
            pl.BlockSpec((1, 3 * D), lambda i: (0, 0)),
            pl.BlockSpec((1, 3 * D), lambda i: (0, 0)),
            pl.BlockSpec((D, D), lambda i: (0, 0)),
            pl.BlockSpec((1, D), lambda i: (0, 0)),
        ],
        out_specs=pl.BlockSpec((256, D), lambda i: (i, 0)),
        out_shape=jax.ShapeDtypeStruct((NPAD, D), f32),
    )(out_gat, den2, x_pad, W_comp, b2(b_gat), b2(b_comp), W_ih, W_hh,
      b2(b_ih), b2(b_hh), W_opt, b2(b_opt))

    return y[:N]


# speedup vs baseline: 26.8180x; 1.0684x over previous
import functools

import jax
import jax.numpy as jnp
from jax import lax
from jax.experimental import pallas as pl
from jax.experimental.pallas import tpu as pltpu
from jax.experimental.pallas import tpu_sc as plsc

N = 10000
E = 320000
D = 128
H = 8

NPAD = 10240
NB = NPAD // 256
ETOT = E + N
C = 128
NS = 16
NCORE = 2
GR = 9
EPAD = 331776
ER = EPAD // 128
TA = EPAD // NS
TR = TA // 128
NGRP = TR // GR
CH_A = EPAD // (NCORE * NS * C)
NROW_T = NPAD // NS


def _head_body(x_ref, wg_ref, vs_ref, vd_ref, xp_ref, us_ref, ud_ref):
    xb = x_ref[...]
    wg = wg_ref[...]
    for h in range(H):
        wh = wg[h * D:(h + 1) * D, :]
        xp_ref[h] = lax.dot_general(
            xb, wh, (((1,), (1,)), ((), ())),
            preferred_element_type=jnp.float32)
    us_ref[...] = jnp.dot(xb, vs_ref[...], preferred_element_type=jnp.float32)
    ud_ref[...] = jnp.dot(xb, vd_ref[...], preferred_element_type=jnp.float32)


def _tail_body(og_ref, den_ref, x_ref, wc_ref, bg_ref, bc_ref, wih_ref,
               whh_ref, bih_ref, bhh_ref, wopt_ref, bopt_ref, o_ref):
    xb = x_ref[...]
    den = den_ref[0] + den_ref[1]
    wc = wc_ref[...]
    bg = bg_ref[...]
    acc = jnp.zeros((256, D), jnp.float32)
    for h in range(H):
        g = og_ref[h] / (den[:, h:h + 1] + 1e-16)
        g = jnp.maximum(g + bg[:, h * D:(h + 1) * D], 0.0)
        acc = acc + lax.dot_general(
            g, wc[:, h * D:(h + 1) * D], (((1,), (1,)), ((), ())),
            preferred_element_type=jnp.float32)
    m = jnp.maximum(acc + bc_ref[...], 0.0)
    gi = lax.dot_general(m, wih_ref[...], (((1,), (1,)), ((), ())),
                         preferred_element_type=jnp.float32) + bih_ref[...]
    gh = lax.dot_general(xb, whh_ref[...], (((1,), (1,)), ((), ())),
                         preferred_element_type=jnp.float32) + bhh_ref[...]
    r = jax.nn.sigmoid(gi[:, 0:D] + gh[:, 0:D])
    z = jax.nn.sigmoid(gi[:, D:2 * D] + gh[:, D:2 * D])
    n = jnp.tanh(gi[:, 2 * D:] + r * gh[:, 2 * D:])
    hv = jnp.tanh((1.0 - z) * n + z * xb)
    o_ref[...] = lax.dot_general(
        hv, wopt_ref[...], (((1,), (1,)), ((), ())),
        preferred_element_type=jnp.float32) + bopt_ref[...]


@functools.partial(
    pl.kernel,
    out_type=(
        jax.ShapeDtypeStruct((EPAD, 16), jnp.float32),
        jax.ShapeDtypeStruct((NCORE, NPAD, 16), jnp.float32),
    ),
    mesh=plsc.VectorSubcoreMesh(
        core_axis_name="c", subcore_axis_name="s",
        num_cores=NCORE, num_subcores=NS),
    compiler_params=pltpu.CompilerParams(use_tc_tiling_on_sc=False),
    scratch_types=[
        pltpu.VMEM((CH_A, 128), jnp.int32),
        pltpu.VMEM((CH_A, 128), jnp.int32),
        pltpu.VMEM((C, 16), jnp.float32),
        pltpu.VMEM((C, 16), jnp.float32),
        pltpu.VMEM((C, 16), jnp.float32),
        pltpu.VMEM((C, 16), jnp.float32),
        pltpu.VMEM((C, 16), jnp.float32),
        pltpu.VMEM((C, 16), jnp.float32),
        pltpu.VMEM_SHARED((NPAD, 16), jnp.float32),
        pltpu.SemaphoreType.DMA,
        pltpu.SemaphoreType.DMA,
        pltpu.SemaphoreType.DMA,
        pltpu.SemaphoreType.DMA,
    ],
)
def _sc_a(us_hbm, ud_hbm, src_hbm, dst_hbm, s_hbm, den_hbm,
          srcb, dstb, us0, us1, ud0, ud1, sv0, sv1, den_sp,
          sg0, sg1, ss0, ss1):
    c = lax.axis_index("c")
    sid = lax.axis_index("s")
    tile_r0 = c * (ER // NCORE) + sid * CH_A
    zrow = sid * NROW_T

    US = (us0, us1)
    UD = (ud0, ud1)
    SV = (sv0, sv1)
    SG = (sg0, sg1)
    SS = (ss0, ss1)

    def _zero_svm(i, carry):
        sv0[i, :] = jnp.zeros((16,), jnp.float32)
        return carry

    lax.fori_loop(0, C, _zero_svm, 0)
    for i in range(NROW_T // C):
        pltpu.sync_copy(sv0, den_sp.at[pl.ds(zrow + i * C, C)])
    plsc.subcore_barrier()

    pltpu.sync_copy(src_hbm.at[pl.ds(tile_r0, CH_A)], srcb)
    pltpu.sync_copy(dst_hbm.at[pl.ds(tile_r0, CH_A)], dstb)

    def _issue(t, p):
        pltpu.async_copy(us_hbm.at[srcb.at[t]], US[p], SG[p])
        pltpu.async_copy(ud_hbm.at[dstb.at[t]], UD[p], SG[p])

    def _consume(t, p):
        pltpu.make_async_copy(us_hbm.at[srcb.at[0]], US[p], SG[p]).wait()
        pltpu.make_async_copy(ud_hbm.at[dstb.at[0]], UD[p], SG[p]).wait()

        @pl.when(t + 1 < CH_A)
        def _():
            _issue(t + 1, 1 - p)

        @pl.when(t >= 2)
        def _():
            pltpu.make_async_copy(
                SV[p], s_hbm.at[pl.ds(0, C)], SS[p]).wait()

        @plsc.parallel_loop(0, C, 1, unroll=2)
        def _edge(e):
            v = US[p][e] + UD[p][e]
            v = jnp.where(v > 0.0, v, 0.2 * v)
            SV[p][e] = jnp.exp(v)

        pltpu.sync_copy(SV[p], den_sp.at[dstb.at[t]], add=True)
        pltpu.make_async_copy(
            SV[p], s_hbm.at[pl.ds((tile_r0 + t) * 128, C)], SS[p]).start()

    _issue(0, 0)

    def _pipe(t, carry):
        @pl.when((t & 1) == 0)
        def _even():
            _consume(t, 0)

        @pl.when((t & 1) == 1)
        def _odd():
            _consume(t, 1)

        return carry

    lax.fori_loop(0, CH_A, _pipe, 0)
    pltpu.make_async_copy(SV[1], s_hbm.at[pl.ds(0, C)], SS[1]).wait()
    pltpu.make_async_copy(SV[0], s_hbm.at[pl.ds(0, C)], SS[0]).wait()
    plsc.subcore_barrier()

    for i in range(NROW_T // C):
        pltpu.sync_copy(den_sp.at[pl.ds(zrow + i * C, C)],
                        den_hbm.at[c, pl.ds(zrow + i * C, C)])


@functools.partial(
    pl.kernel,
    out_type=jax.ShapeDtypeStruct((H, NPAD, D), jnp.float32),
    mesh=plsc.VectorSubcoreMesh(
        core_axis_name="c", subcore_axis_name="s",
        num_cores=NCORE, num_subcores=NS),
    compiler_params=pltpu.CompilerParams(use_tc_tiling_on_sc=False),
    scratch_types=[
        pltpu.VMEM((C, D), jnp.float32),
        pltpu.VMEM((C, D), jnp.float32),
        pltpu.VMEM((C, 16), jnp.float32),
        pltpu.VMEM((C, 16), jnp.float32),
        pltpu.VMEM((GR, 128), jnp.int32),
        pltpu.VMEM((GR, 128), jnp.int32),
        pltpu.VMEM((GR, 128), jnp.int32),
        pltpu.VMEM((GR, 128), jnp.int32),
        pltpu.VMEM((1, 128), jnp.int32),
        pltpu.VMEM((1, 128), jnp.int32),
        pltpu.VMEM_SHARED((NPAD, D), jnp.float32),
        pltpu.SemaphoreType.DMA,
        pltpu.SemaphoreType.DMA,
        pltpu.SemaphoreType.DMA,
        pltpu.SemaphoreType.DMA,
        pltpu.SemaphoreType.DMA,
        pltpu.SemaphoreType.DMA,
        pltpu.SemaphoreType.DMA,
    ],
)
def _sc_b(xp_hbm, src_hbm, dst_hbm, s_hbm, outg_hbm,
          rows_a, rows_b, s_a, s_b, srcg_a, srcg_b, dstg_a, dstg_b,
          idx_a, idx_b, out_sp,
          sem_x0, sem_x1, sem_s0, sem_s1, sem_w0, sem_w1, sem_g):
    c = lax.axis_index("c")
    sid = lax.axis_index("s")
    tile_r0 = sid * TR
    zrow = sid * NROW_T

    RO = (rows_a, rows_b)
    SB = (s_a, s_b)
    IX = (idx_a, idx_b)
    SX = (sem_x0, sem_x1)
    SS = (sem_s0, sem_s1)
    SW = (sem_w0, sem_w1)
    GRP = ((srcg_a, dstg_a), (srcg_b, dstg_b))

    def _zero_rows_a(i, carry):
        for j in range(8):
            rows_a[i, pl.ds(j * 16, 16)] = jnp.zeros((16,), jnp.float32)
        return carry

    def _issue_grp(g, gp):
        srcg, dstg = GRP[gp]
        r0 = tile_r0 + g * GR
        pltpu.async_copy(src_hbm.at[pl.ds(r0, GR)], srcg, sem_g)
        pltpu.async_copy(dst_hbm.at[pl.ds(r0, GR)], dstg, sem_g)

    def _wait_grp(gp):
        srcg, dstg = GRP[gp]
        pltpu.make_async_copy(src_hbm.at[pl.ds(0, GR)], srcg, sem_g).wait()
        pltpu.make_async_copy(dst_hbm.at[pl.ds(0, GR)], dstg, sem_g).wait()

    for hh in range(H // NCORE):
        h = c * (H // NCORE) + hh
        hoff = h * NPAD

        lax.fori_loop(0, C, _zero_rows_a, 0)
        for i in range(NROW_T // C):
            pltpu.sync_copy(rows_a, out_sp.at[pl.ds(zrow + i * C, C)])
        plsc.subcore_barrier()

        def _fill_idx(b, gp, k):
            srcg, _ = GRP[gp]
            for j in range(8):
                IX[b][0, pl.ds(j * 16, 16)] = (
                    srcg[k, pl.ds(j * 16, 16)] + hoff)

        def _issue_chunk(b, g, gp, k):
            _fill_idx(b, gp, k)
            pltpu.async_copy(xp_hbm.at[IX[b].at[0]], RO[b], SX[b])
            e0 = (tile_r0 + g * GR + k) * 128
            pltpu.async_copy(s_hbm.at[pl.ds(e0, C)], SB[b], SS[b])

        def _drain_scatter(b):
            _, dstg = GRP[0]
            pltpu.make_async_copy(
                RO[b], out_sp.at[dstg.at[0]], SW[b]).wait()

        def _consume(g, gp, k):
            b = (gp * GR + k) % 2
            nb = 1 - b
            if k > 0:
                _drain_scatter(nb)
            pltpu.make_async_copy(xp_hbm.at[IX[b].at[0]], RO[b], SX[b]).wait()
            pltpu.make_async_copy(s_hbm.at[pl.ds(0, C)], SB[b], SS[b]).wait()
            if k < GR - 1:
                _issue_chunk(nb, g, gp, k + 1)
            else:
                @pl.when(g < NGRP - 1)
                def _():
                    _wait_grp(1 - gp)
                    _issue_chunk(nb, g + 1, 1 - gp, 0)

            @plsc.parallel_loop(0, C, 1, unroll=2)
            def _edge(e):
                sv = SB[b][e]
                bc = lax.gather(
                    sv, jnp.full((16, 1), h, jnp.int32),
                    lax.GatherDimensionNumbers(
                        offset_dims=(), collapsed_slice_dims=(0,),
                        start_index_map=(0,)),
                    slice_sizes=(1,),
                    mode=lax.GatherScatterMode.PROMISE_IN_BOUNDS)
                for j in range(8):
                    RO[b][e, pl.ds(j * 16, 16)] = (
                        RO[b][e, pl.ds(j * 16, 16)] * bc)

            _, dstg = GRP[gp]
            sc = pltpu.make_async_copy(RO[b], out_sp.at[dstg.at[k]], SW[b])
            sc.start(add=True)
            if k == GR - 1:
                sc.wait()

                @pl.when(g < NGRP - 2)
                def _():
                    _issue_grp(g + 2, gp)

        _issue_grp(0, 0)
        _wait_grp(0)
        _issue_grp(1, 1)
        _issue_chunk(0, 0, 0, 0)

        def _group_body(g, carry):
            @pl.when((g & 1) == 0)
            def _even():
                for k in range(GR):
                    _consume(g, 0, k)

            @pl.when((g & 1) == 1)
            def _odd():
                for k in range(GR):
                    _consume(g, 1, k)

            return carry

        lax.fori_loop(0, NGRP, _group_body, 0)
        plsc.subcore_barrier()
        for i in range(NROW_T // C):
            pltpu.sync_copy(out_sp.at[pl.ds(zrow + i * C, C)],
                            outg_hbm.at[h, pl.ds(zrow + i * C, C)])
        plsc.subcore_barrier()


def kernel(x, edge_index, W_gat, att_src, att_dst, b_gat, W_comp, b_comp,
           W_ih, W_hh, b_ih, b_hh, W_opt, b_opt):
    f32 = jnp.float32
    x_pad = jnp.zeros((NPAD, D), f32).at[:N].set(x)

    wg3 = W_gat.reshape(H, D, D)
    vsrc = jnp.einsum("hdk,hd->kh", wg3, att_src)
    vdst = jnp.einsum("hdk,hd->kh", wg3, att_dst)
    vsrc2 = jnp.concatenate([vsrc, vsrc], axis=1)
    vdst2 = jnp.concatenate([vdst, vdst], axis=1)

    loop = jnp.arange(N, dtype=jnp.int32)
    src = jnp.concatenate([edge_index[0].astype(jnp.int32), loop])
    dst = jnp.concatenate([edge_index[1].astype(jnp.int32), loop])
    pad = jnp.full((EPAD - ETOT,), N, jnp.int32)
    src2d = jnp.concatenate([src, pad]).reshape(ER, 128)
    dst2d = jnp.concatenate([dst, pad]).reshape(ER, 128)

    xp_t, us, ud = pl.pallas_call(
        _head_body,
        grid=(NB,),
        in_specs=[
            pl.BlockSpec((256, D), lambda i: (i, 0)),
            pl.BlockSpec((H * D, D), lambda i: (0, 0)),
            pl.BlockSpec((D, 16), lambda i: (0, 0)),
            pl.BlockSpec((D, 16), lambda i: (0, 0)),
        ],
        out_specs=[
            pl.BlockSpec((H, 256, D), lambda i: (0, i, 0)),
            pl.BlockSpec((256, 16), lambda i: (i, 0)),
            pl.BlockSpec((256, 16), lambda i: (i, 0)),
        ],
        out_shape=[
            jax.ShapeDtypeStruct((H, NPAD, D), f32),
            jax.ShapeDtypeStruct((NPAD, 16), f32),
            jax.ShapeDtypeStruct((NPAD, 16), f32),
        ],
    )(x_pad, W_gat, vsrc2, vdst2)

    s_buf, den2 = _sc_a(us, ud, src2d, dst2d)
    xp_flat = xp_t.reshape(H * NPAD, D)
    out_gat = _sc_b(xp_flat, src2d, dst2d, s_buf)

    b2 = lambda b: b.reshape(1, -1)
    y = pl.pallas_call(
        _tail_body,
        grid=(NB,),
        in_specs=[
            pl.BlockSpec((H, 256, D), lambda i: (0, i, 0)),
            pl.BlockSpec((NCORE, 256, 16), lambda i: (0, i, 0)),
            pl.BlockSpec((256, D), lambda i: (i, 0)),
            pl.BlockSpec((D, H * D), lambda i: (0, 0)),
            pl.BlockSpec((1, H * D), lambda i: (0, 0)),
            pl.BlockSpec((1, D), lambda i: (0, 0)),
            pl.BlockSpec((3 * D, D), lambda i: (0, 0)),
            pl.BlockSpec((3 * D, D), lambda i: (0, 0)),
            pl.BlockSpec((1, 3 * D), lambda i: (0, 0)),
            pl.BlockSpec((1, 3 * D), lambda i: (0, 0)),
            pl.BlockSpec((D, D), lambda i: (0, 0)),
            pl.BlockSpec((1, D), lambda i: (0, 0)),
        ],
        out_specs=pl.BlockSpec((256, D), lambda i: (i, 0)),
        out_shape=jax.ShapeDtypeStruct((NPAD, D), f32),
    )(out_gat, den2, x_pad, W_comp, b2(b_gat), b2(b_comp), W_ih, W_hh,
      b2(b_ih), b2(b_hh), W_opt, b2(b_opt))

    return y[:N]
